# Initial kernel scaffold; baseline (speedup 1.0000x reference)
import jax
import jax.numpy as jnp
from jax.experimental import pallas as pl


def kernel(input_feat, pos_table, ln_gamma, ln_beta):
    raise NotImplementedError("write your pallas kernel here")


# speedup vs baseline: 3.5407x; 3.5407x over previous
import jax
import jax.numpy as jnp
from jax.experimental import pallas as pl
from jax.experimental.pallas import tpu as pltpu

_EPS = 1e-5
_S_BLK = 512


def _ln_body(x_ref, pos_ref, g_ref, b_ref, o_ref):
    x = x_ref[...]
    p = pos_ref[...]
    e = x + p[None, :, :]
    mean = jnp.mean(e, axis=-1, keepdims=True)
    c = e - mean
    var = jnp.mean(c * c, axis=-1, keepdims=True)
    o_ref[...] = c * jax.lax.rsqrt(var + _EPS) * g_ref[...] + b_ref[...]


def kernel(input_feat, pos_table, ln_gamma, ln_beta):
    bsz, seq, d = input_feat.shape
    n_s = seq // _S_BLK
    grid = (n_s, bsz)
    return pl.pallas_call(
        _ln_body,
        grid=grid,
        in_specs=[
            pl.BlockSpec((1, _S_BLK, d), lambda i, j: (j, i, 0)),
            pl.BlockSpec((_S_BLK, d), lambda i, j: (i, 0)),
            pl.BlockSpec((d,), lambda i, j: (0,)),
            pl.BlockSpec((d,), lambda i, j: (0,)),
        ],
        out_specs=pl.BlockSpec((1, _S_BLK, d), lambda i, j: (j, i, 0)),
        out_shape=jax.ShapeDtypeStruct((bsz, seq, d), input_feat.dtype),
        compiler_params=pltpu.CompilerParams(
            dimension_semantics=("arbitrary", "arbitrary"),
        ),
    )(input_feat, pos_table, ln_gamma, ln_beta)


# speedup vs baseline: 4.0265x; 1.1372x over previous
import jax
import jax.numpy as jnp
from jax.experimental import pallas as pl
from jax.experimental.pallas import tpu as pltpu

_EPS = 1e-5
_S_BLK = 1024


def _ln_body(x_ref, pos_ref, g_ref, b_ref, o_ref):
    x = x_ref[...]
    p = pos_ref[...]
    e = x + p[None, :, :]
    mean = jnp.mean(e, axis=-1, keepdims=True)
    c = e - mean
    var = jnp.mean(c * c, axis=-1, keepdims=True)
    o_ref[...] = c * jax.lax.rsqrt(var + _EPS) * g_ref[...] + b_ref[...]


def kernel(input_feat, pos_table, ln_gamma, ln_beta):
    bsz, seq, d = input_feat.shape
    n_s = seq // _S_BLK
    grid = (n_s, bsz)
    return pl.pallas_call(
        _ln_body,
        grid=grid,
        in_specs=[
            pl.BlockSpec((1, _S_BLK, d), lambda i, j: (j, i, 0)),
            pl.BlockSpec((_S_BLK, d), lambda i, j: (i, 0)),
            pl.BlockSpec((d,), lambda i, j: (0,)),
            pl.BlockSpec((d,), lambda i, j: (0,)),
        ],
        out_specs=pl.BlockSpec((1, _S_BLK, d), lambda i, j: (j, i, 0)),
        out_shape=jax.ShapeDtypeStruct((bsz, seq, d), input_feat.dtype),
        compiler_params=pltpu.CompilerParams(
            dimension_semantics=("arbitrary", "arbitrary"),
        ),
    )(input_feat, pos_table, ln_gamma, ln_beta)


# speedup vs baseline: 4.2327x; 1.0512x over previous
import jax
import jax.numpy as jnp
from jax.experimental import pallas as pl
from jax.experimental.pallas import tpu as pltpu

_EPS = 1e-5
_S_BLK = 2048


def _ln_body(x_ref, pos_ref, g_ref, b_ref, o_ref):
    x = x_ref[...]
    p = pos_ref[...]
    e = x + p[None, :, :]
    mean = jnp.mean(e, axis=-1, keepdims=True)
    c = e - mean
    var = jnp.mean(c * c, axis=-1, keepdims=True)
    o_ref[...] = c * jax.lax.rsqrt(var + _EPS) * g_ref[...] + b_ref[...]


def kernel(input_feat, pos_table, ln_gamma, ln_beta):
    bsz, seq, d = input_feat.shape
    n_s = seq // _S_BLK
    grid = (n_s, bsz)
    return pl.pallas_call(
        _ln_body,
        grid=grid,
        in_specs=[
            pl.BlockSpec((1, _S_BLK, d), lambda i, j: (j, i, 0)),
            pl.BlockSpec((_S_BLK, d), lambda i, j: (i, 0)),
            pl.BlockSpec((d,), lambda i, j: (0,)),
            pl.BlockSpec((d,), lambda i, j: (0,)),
        ],
        out_specs=pl.BlockSpec((1, _S_BLK, d), lambda i, j: (j, i, 0)),
        out_shape=jax.ShapeDtypeStruct((bsz, seq, d), input_feat.dtype),
        compiler_params=pltpu.CompilerParams(
            dimension_semantics=("arbitrary", "arbitrary"),
        ),
    )(input_feat, pos_table, ln_gamma, ln_beta)
